# baseline (speedup 1.0000x reference)
import functools

import jax
import jax.numpy as jnp
from jax import lax
from jax.experimental import pallas as pl
from jax.experimental.pallas import tpu as pltpu
from jax.experimental.pallas import tpu_sc as plsc

N = 10000
E = 320000
D = 128

NC = 2
NS = 16
NW = NC * NS
CHUNK = 128
CF = 125
CS = 33
FAST_CID = 0
CHM = CF
EP = NW * CHM * CHUNK
N_ACC = 10112
ZROWS = N_ACC // NS
ZTAIL = ZROWS - 4 * CHUNK
_SIZES = [CHUNK * (CF if (w % NC) == FAST_CID else CS) for w in range(NW)]
_EPAD = sum(_SIZES)


def _sc_neighbor_sum(g, row3, col3):
  mesh = plsc.VectorSubcoreMesh(core_axis_name="c", subcore_axis_name="s")

  @functools.partial(
      pl.kernel,
      out_type=jax.ShapeDtypeStruct((NC, N_ACC, D), jnp.float32),
      mesh=mesh,
      scratch_types=[
          pltpu.VMEM((CHM, CHUNK), jnp.int32),
          pltpu.VMEM((CHUNK,), jnp.int32),
          pltpu.VMEM((CHUNK,), jnp.int32),
          pltpu.VMEM((CHUNK, D), jnp.float32),
          pltpu.VMEM((CHUNK, D), jnp.float32),
          pltpu.VMEM_SHARED((N_ACC, D), jnp.float32),
          pltpu.SemaphoreType.DMA,
          pltpu.SemaphoreType.DMA,
          pltpu.SemaphoreType.DMA,
          pltpu.SemaphoreType.DMA,
      ],
  )
  def k(g_hbm, row_hbm, col_hbm, out_hbm, rowv, cb, cb2, buf, buf2, acc,
        sem, sem2, csem, csem2):
    cid = lax.axis_index("c")
    sid = lax.axis_index("s")
    wid = sid * NC + cid
    nch = jnp.where(cid == FAST_CID, CF, CS)

    pltpu.sync_copy(row_hbm.at[wid], rowv)

    @pl.loop(0, CHUNK)
    def _(r):
      @pl.loop(0, D, step=16)
      def _(c):
        buf[r, pl.ds(c, 16)] = jnp.zeros((16,), jnp.float32)

    @pl.loop(0, 4)
    def _(z):
      pltpu.sync_copy(buf, acc.at[pl.ds(sid * ZROWS + z * CHUNK, CHUNK)])
    pltpu.sync_copy(buf.at[pl.ds(0, ZTAIL)],
                    acc.at[pl.ds(sid * ZROWS + 4 * CHUNK, ZTAIL)])

    plsc.subcore_barrier()

    @pl.when(nch > 0)
    def _():
      pltpu.async_copy(g_hbm.at[rowv.at[0]], buf, sem)
      pltpu.async_copy(col_hbm.at[wid, 0], cb, csem)

    @pl.loop(0, nch, step=2)
    def _(j):
      @pl.when(j + 1 < nch)
      def _():
        pltpu.async_copy(g_hbm.at[rowv.at[j + 1]], buf2, sem2)
        pltpu.async_copy(col_hbm.at[wid, j + 1], cb2, csem2)

      pltpu.make_async_copy(g_hbm.at[rowv.at[j]], buf, sem).wait()
      pltpu.make_async_copy(col_hbm.at[wid, j], cb, csem).wait()
      pltpu.sync_copy(buf, acc.at[cb], add=True)

      @pl.when(j + 2 < nch)
      def _():
        pltpu.async_copy(g_hbm.at[rowv.at[j + 2]], buf, sem)
        pltpu.async_copy(col_hbm.at[wid, j + 2], cb, csem)

      @pl.when(j + 1 < nch)
      def _():
        pltpu.make_async_copy(g_hbm.at[rowv.at[j + 1]], buf2, sem2).wait()
        pltpu.make_async_copy(col_hbm.at[wid, j + 1], cb2, csem2).wait()
        pltpu.sync_copy(buf2, acc.at[cb2], add=True)

    plsc.subcore_barrier()

    @pl.loop(0, 4)
    def _(z):
      b = sid * ZROWS + z * CHUNK
      pltpu.sync_copy(acc.at[pl.ds(b, CHUNK)], out_hbm.at[cid, pl.ds(b, CHUNK)])
    bt = sid * ZROWS + 4 * CHUNK
    pltpu.sync_copy(acc.at[pl.ds(bt, ZTAIL)], out_hbm.at[cid, pl.ds(bt, ZTAIL)])

  return k(g, row3, col3)


_BR = 2000
_GRID = N // _BR


def _mm_body(x_ref, w_ref, o_ref):
  o_ref[...] = jnp.dot(x_ref[...], w_ref[...],
                       preferred_element_type=jnp.float32)


def _fuse_body(g_ref, p_ref, b_ref, w_ref, o_ref):
  h = g_ref[...] + p_ref[0] + p_ref[1] + b_ref[...]
  h = jnp.maximum(h, 0.0)
  o_ref[...] = jnp.dot(h, w_ref[...], preferred_element_type=jnp.float32)


def _final_body(g_ref, p_ref, b_ref, w_ref, bo_ref, o_ref):
  h = g_ref[...] + p_ref[0] + p_ref[1] + b_ref[...]
  h = jnp.maximum(h, 0.0)
  t = jnp.dot(h, w_ref[...], preferred_element_type=jnp.float32) + bo_ref[...]
  m = jnp.max(t, axis=1, keepdims=True)
  e = t - m
  o_ref[...] = e - jnp.log(jnp.sum(jnp.exp(e), axis=1, keepdims=True))


def _tc_matmul(x, w):
  return pl.pallas_call(
      _mm_body,
      grid=(_GRID,),
      in_specs=[
          pl.BlockSpec((_BR, D), lambda i: (i, 0)),
          pl.BlockSpec((D, D), lambda i: (0, 0)),
      ],
      out_specs=pl.BlockSpec((_BR, D), lambda i: (i, 0)),
      out_shape=jax.ShapeDtypeStruct((N, D), jnp.float32),
  )(x, w)


def _tc_fuse_matmul(g, p, b, w):
  return pl.pallas_call(
      _fuse_body,
      grid=(_GRID,),
      in_specs=[
          pl.BlockSpec((_BR, D), lambda i: (i, 0)),
          pl.BlockSpec((2, _BR, D), lambda i: (0, i, 0)),
          pl.BlockSpec((1, D), lambda i: (0, 0)),
          pl.BlockSpec((D, D), lambda i: (0, 0)),
      ],
      out_specs=pl.BlockSpec((_BR, D), lambda i: (i, 0)),
      out_shape=jax.ShapeDtypeStruct((N, D), jnp.float32),
  )(g, p, b, w)


def _tc_final(g, p, b, w, bo):
  return pl.pallas_call(
      _final_body,
      grid=(_GRID,),
      in_specs=[
          pl.BlockSpec((_BR, D), lambda i: (i, 0)),
          pl.BlockSpec((2, _BR, D), lambda i: (0, i, 0)),
          pl.BlockSpec((1, D), lambda i: (0, 0)),
          pl.BlockSpec((D, D), lambda i: (0, 0)),
          pl.BlockSpec((1, D), lambda i: (0, 0)),
      ],
      out_specs=pl.BlockSpec((_BR, D), lambda i: (i, 0)),
      out_shape=jax.ShapeDtypeStruct((N, D), jnp.float32),
  )(g, p, b, w, bo)


@jax.jit
def kernel(x, edge_index, W1, b1, W2, b2, Wo, bo):
  row = edge_index[0]
  col = edge_index[1]
  rowp = jnp.pad(row, (0, _EPAD - E))
  colp = jnp.pad(col, (0, _EPAD - E), constant_values=N)
  row_parts = []
  col_parts = []
  off = 0
  for w in range(NW):
    sz = _SIZES[w]
    tail = CHM * CHUNK - sz
    row_parts.append(jnp.pad(lax.slice(rowp, (off,), (off + sz,)), (0, tail)))
    col_parts.append(jnp.pad(lax.slice(colp, (off,), (off + sz,)), (0, tail),
                             constant_values=N))
    off += sz
  row3 = jnp.stack(row_parts).reshape(NW, CHM, CHUNK)
  col3 = jnp.stack(col_parts).reshape(NW, CHM, CHUNK)

  b1r = b1.reshape(1, D)
  b2r = b2.reshape(1, D)
  bor = bo.reshape(1, D)

  g1 = _tc_matmul(x, W1)
  p1 = _sc_neighbor_sum(g1, row3, col3)
  g2 = _tc_fuse_matmul(g1, p1, b1r, W2)
  p2 = _sc_neighbor_sum(g2, row3, col3)
  return _tc_final(g2, p2, b2r, Wo, bor)


# speedup vs baseline: 1.0102x; 1.0102x over previous
import functools

import jax
import jax.numpy as jnp
from jax import lax
from jax.experimental import pallas as pl
from jax.experimental.pallas import tpu as pltpu
from jax.experimental.pallas import tpu_sc as plsc

N = 10000
E = 320000
D = 128

NC = 2
NS = 16
NW = NC * NS
CHUNK = 128
CF = 118
CS = 40
FAST_CID = 0
CHM = CF
EP = NW * CHM * CHUNK
N_ACC = 10240
ZROWS = N_ACC // NS
ZITER = ZROWS // CHUNK
_SIZES = [CHUNK * (CF if (w % NC) == FAST_CID else CS) for w in range(NW)]
_EPAD = sum(_SIZES)


def _sc_neighbor_sum(g, row3, col3):
  mesh = plsc.VectorSubcoreMesh(core_axis_name="c", subcore_axis_name="s")

  @functools.partial(
      pl.kernel,
      out_type=jax.ShapeDtypeStruct((NC, N_ACC, D), jnp.float32),
      mesh=mesh,
      scratch_types=[
          pltpu.VMEM((CHM, CHUNK), jnp.int32),
          pltpu.VMEM((CHUNK,), jnp.int32),
          pltpu.VMEM((CHUNK,), jnp.int32),
          pltpu.VMEM((CHUNK, D), jnp.float32),
          pltpu.VMEM((CHUNK, D), jnp.float32),
          pltpu.VMEM_SHARED((N_ACC, D), jnp.float32),
          pltpu.SemaphoreType.DMA,
          pltpu.SemaphoreType.DMA,
          pltpu.SemaphoreType.DMA,
          pltpu.SemaphoreType.DMA,
      ],
  )
  def k(g_hbm, row_hbm, col_hbm, out_hbm, rowv, cb, cb2, buf, buf2, acc,
        sem, sem2, csem, csem2):
    cid = lax.axis_index("c")
    sid = lax.axis_index("s")
    wid = sid * NC + cid
    nch = jnp.where(cid == FAST_CID, CF, CS)

    pltpu.sync_copy(row_hbm.at[wid], rowv)

    @pl.loop(0, CHUNK)
    def _(r):
      @pl.loop(0, D, step=16)
      def _(c):
        buf[r, pl.ds(c, 16)] = jnp.zeros((16,), jnp.float32)

    @pl.loop(0, ZITER)
    def _(z):
      pltpu.sync_copy(buf, acc.at[pl.ds(sid * ZROWS + z * CHUNK, CHUNK)])

    plsc.subcore_barrier()

    @pl.when(nch > 0)
    def _():
      pltpu.async_copy(g_hbm.at[rowv.at[0]], buf, sem)
      pltpu.async_copy(col_hbm.at[wid, 0], cb, csem)

    @pl.loop(0, nch, step=2)
    def _(j):
      @pl.when(j + 1 < nch)
      def _():
        pltpu.async_copy(g_hbm.at[rowv.at[j + 1]], buf2, sem2)
        pltpu.async_copy(col_hbm.at[wid, j + 1], cb2, csem2)

      pltpu.make_async_copy(g_hbm.at[rowv.at[j]], buf, sem).wait()
      pltpu.make_async_copy(col_hbm.at[wid, j], cb, csem).wait()
      pltpu.sync_copy(buf, acc.at[cb], add=True)

      @pl.when(j + 2 < nch)
      def _():
        pltpu.async_copy(g_hbm.at[rowv.at[j + 2]], buf, sem)
        pltpu.async_copy(col_hbm.at[wid, j + 2], cb, csem)

      @pl.when(j + 1 < nch)
      def _():
        pltpu.make_async_copy(g_hbm.at[rowv.at[j + 1]], buf2, sem2).wait()
        pltpu.make_async_copy(col_hbm.at[wid, j + 1], cb2, csem2).wait()
        pltpu.sync_copy(buf2, acc.at[cb2], add=True)

    plsc.subcore_barrier()

    @pl.loop(0, ZITER)
    def _(z):
      b = sid * ZROWS + z * CHUNK
      pltpu.sync_copy(acc.at[pl.ds(b, CHUNK)], out_hbm.at[cid, pl.ds(b, CHUNK)])

  return k(g, row3, col3)


_BR = 2000
_GRID = N // _BR


def _mm_body(x_ref, w_ref, o_ref):
  o_ref[...] = jnp.dot(x_ref[...], w_ref[...],
                       preferred_element_type=jnp.float32)


def _fuse_body(g_ref, p_ref, b_ref, w_ref, o_ref):
  h = g_ref[...] + p_ref[0] + p_ref[1] + b_ref[...]
  h = jnp.maximum(h, 0.0)
  o_ref[...] = jnp.dot(h, w_ref[...], preferred_element_type=jnp.float32)


def _final_body(g_ref, p_ref, b_ref, w_ref, bo_ref, o_ref):
  h = g_ref[...] + p_ref[0] + p_ref[1] + b_ref[...]
  h = jnp.maximum(h, 0.0)
  t = jnp.dot(h, w_ref[...], preferred_element_type=jnp.float32) + bo_ref[...]
  m = jnp.max(t, axis=1, keepdims=True)
  e = t - m
  o_ref[...] = e - jnp.log(jnp.sum(jnp.exp(e), axis=1, keepdims=True))


def _tc_matmul(x, w):
  return pl.pallas_call(
      _mm_body,
      grid=(_GRID,),
      in_specs=[
          pl.BlockSpec((_BR, D), lambda i: (i, 0)),
          pl.BlockSpec((D, D), lambda i: (0, 0)),
      ],
      out_specs=pl.BlockSpec((_BR, D), lambda i: (i, 0)),
      out_shape=jax.ShapeDtypeStruct((N, D), jnp.float32),
  )(x, w)


def _tc_fuse_matmul(g, p, b, w):
  return pl.pallas_call(
      _fuse_body,
      grid=(_GRID,),
      in_specs=[
          pl.BlockSpec((_BR, D), lambda i: (i, 0)),
          pl.BlockSpec((2, _BR, D), lambda i: (0, i, 0)),
          pl.BlockSpec((1, D), lambda i: (0, 0)),
          pl.BlockSpec((D, D), lambda i: (0, 0)),
      ],
      out_specs=pl.BlockSpec((_BR, D), lambda i: (i, 0)),
      out_shape=jax.ShapeDtypeStruct((N, D), jnp.float32),
  )(g, p, b, w)


def _tc_final(g, p, b, w, bo):
  return pl.pallas_call(
      _final_body,
      grid=(_GRID,),
      in_specs=[
          pl.BlockSpec((_BR, D), lambda i: (i, 0)),
          pl.BlockSpec((2, _BR, D), lambda i: (0, i, 0)),
          pl.BlockSpec((1, D), lambda i: (0, 0)),
          pl.BlockSpec((D, D), lambda i: (0, 0)),
          pl.BlockSpec((1, D), lambda i: (0, 0)),
      ],
      out_specs=pl.BlockSpec((_BR, D), lambda i: (i, 0)),
      out_shape=jax.ShapeDtypeStruct((N, D), jnp.float32),
  )(g, p, b, w, bo)


@jax.jit
def kernel(x, edge_index, W1, b1, W2, b2, Wo, bo):
  row = edge_index[0]
  col = edge_index[1]
  rowp = jnp.pad(row, (0, _EPAD - E))
  colp = jnp.pad(col, (0, _EPAD - E), constant_values=N)
  row_parts = []
  col_parts = []
  off = 0
  for w in range(NW):
    sz = _SIZES[w]
    tail = CHM * CHUNK - sz
    row_parts.append(jnp.pad(lax.slice(rowp, (off,), (off + sz,)), (0, tail)))
    col_parts.append(jnp.pad(lax.slice(colp, (off,), (off + sz,)), (0, tail),
                             constant_values=N))
    off += sz
  row3 = jnp.stack(row_parts).reshape(NW, CHM, CHUNK)
  col3 = jnp.stack(col_parts).reshape(NW, CHM, CHUNK)

  b1r = b1.reshape(1, D)
  b2r = b2.reshape(1, D)
  bor = bo.reshape(1, D)

  g1 = _tc_matmul(x, W1)
  p1 = _sc_neighbor_sum(g1, row3, col3)
  g2 = _tc_fuse_matmul(g1, p1, b1r, W2)
  p2 = _sc_neighbor_sum(g2, row3, col3)
  return _tc_final(g2, p2, b2r, Wo, bor)


# speedup vs baseline: 1.0328x; 1.0225x over previous
import functools

import jax
import jax.numpy as jnp
from jax import lax
from jax.experimental import pallas as pl
from jax.experimental.pallas import tpu as pltpu
from jax.experimental.pallas import tpu_sc as plsc

N = 10000
E = 320000
D = 128

NC = 2
NS = 16
NW = NC * NS
CHUNK = 128
CF = 120
CS = 38
FAST_CID = 0
CHM = CF
EP = NW * CHM * CHUNK
N_ACC = 10240
ZROWS = N_ACC // NS
ZITER = ZROWS // CHUNK
_SIZES = [CHUNK * (CF if (w % NC) == FAST_CID else CS) for w in range(NW)]
_EPAD = sum(_SIZES)


def _sc_neighbor_sum(g, row3, col3):
  mesh = plsc.VectorSubcoreMesh(core_axis_name="c", subcore_axis_name="s")

  @functools.partial(
      pl.kernel,
      out_type=jax.ShapeDtypeStruct((NC, N_ACC, D), jnp.float32),
      mesh=mesh,
      scratch_types=[
          pltpu.VMEM((CHM, CHUNK), jnp.int32),
          pltpu.VMEM((CHUNK,), jnp.int32),
          pltpu.VMEM((CHUNK,), jnp.int32),
          pltpu.VMEM((CHUNK, D), jnp.float32),
          pltpu.VMEM((CHUNK, D), jnp.float32),
          pltpu.VMEM_SHARED((N_ACC, D), jnp.float32),
          pltpu.SemaphoreType.DMA,
          pltpu.SemaphoreType.DMA,
          pltpu.SemaphoreType.DMA,
          pltpu.SemaphoreType.DMA,
      ],
  )
  def k(g_hbm, row_hbm, col_hbm, out_hbm, rowv, cb, cb2, buf, buf2, acc,
        sem, sem2, csem, csem2):
    cid = lax.axis_index("c")
    sid = lax.axis_index("s")
    wid = sid * NC + cid
    nch = jnp.where(cid == FAST_CID, CF, CS)

    pltpu.sync_copy(row_hbm.at[wid], rowv)

    @pl.loop(0, CHUNK)
    def _(r):
      @pl.loop(0, D, step=16)
      def _(c):
        buf[r, pl.ds(c, 16)] = jnp.zeros((16,), jnp.float32)

    @pl.loop(0, ZITER)
    def _(z):
      pltpu.sync_copy(buf, acc.at[pl.ds(sid * ZROWS + z * CHUNK, CHUNK)])

    plsc.subcore_barrier()

    @pl.when(nch > 0)
    def _():
      pltpu.async_copy(g_hbm.at[rowv.at[0]], buf, sem)
      pltpu.async_copy(col_hbm.at[wid, 0], cb, csem)

    @pl.loop(0, nch, step=2)
    def _(j):
      @pl.when(j + 1 < nch)
      def _():
        pltpu.async_copy(g_hbm.at[rowv.at[j + 1]], buf2, sem2)
        pltpu.async_copy(col_hbm.at[wid, j + 1], cb2, csem2)

      pltpu.make_async_copy(g_hbm.at[rowv.at[j]], buf, sem).wait()
      pltpu.make_async_copy(col_hbm.at[wid, j], cb, csem).wait()
      pltpu.sync_copy(buf, acc.at[cb], add=True)

      @pl.when(j + 2 < nch)
      def _():
        pltpu.async_copy(g_hbm.at[rowv.at[j + 2]], buf, sem)
        pltpu.async_copy(col_hbm.at[wid, j + 2], cb, csem)

      @pl.when(j + 1 < nch)
      def _():
        pltpu.make_async_copy(g_hbm.at[rowv.at[j + 1]], buf2, sem2).wait()
        pltpu.make_async_copy(col_hbm.at[wid, j + 1], cb2, csem2).wait()
        pltpu.sync_copy(buf2, acc.at[cb2], add=True)

    plsc.subcore_barrier()

    @pl.loop(0, ZITER)
    def _(z):
      b = sid * ZROWS + z * CHUNK
      pltpu.sync_copy(acc.at[pl.ds(b, CHUNK)], out_hbm.at[cid, pl.ds(b, CHUNK)])

  return k(g, row3, col3)


_BR = 2000
_GRID = N // _BR


def _mm_body(x_ref, w_ref, o_ref):
  o_ref[...] = jnp.dot(x_ref[...], w_ref[...],
                       preferred_element_type=jnp.float32)


def _fuse_body(g_ref, p_ref, b_ref, w_ref, o_ref):
  h = g_ref[...] + p_ref[0] + p_ref[1] + b_ref[...]
  h = jnp.maximum(h, 0.0)
  o_ref[...] = jnp.dot(h, w_ref[...], preferred_element_type=jnp.float32)


def _final_body(g_ref, p_ref, b_ref, w_ref, bo_ref, o_ref):
  h = g_ref[...] + p_ref[0] + p_ref[1] + b_ref[...]
  h = jnp.maximum(h, 0.0)
  t = jnp.dot(h, w_ref[...], preferred_element_type=jnp.float32) + bo_ref[...]
  m = jnp.max(t, axis=1, keepdims=True)
  e = t - m
  o_ref[...] = e - jnp.log(jnp.sum(jnp.exp(e), axis=1, keepdims=True))


def _tc_matmul(x, w):
  return pl.pallas_call(
      _mm_body,
      grid=(_GRID,),
      in_specs=[
          pl.BlockSpec((_BR, D), lambda i: (i, 0)),
          pl.BlockSpec((D, D), lambda i: (0, 0)),
      ],
      out_specs=pl.BlockSpec((_BR, D), lambda i: (i, 0)),
      out_shape=jax.ShapeDtypeStruct((N, D), jnp.float32),
  )(x, w)


def _tc_fuse_matmul(g, p, b, w):
  return pl.pallas_call(
      _fuse_body,
      grid=(_GRID,),
      in_specs=[
          pl.BlockSpec((_BR, D), lambda i: (i, 0)),
          pl.BlockSpec((2, _BR, D), lambda i: (0, i, 0)),
          pl.BlockSpec((1, D), lambda i: (0, 0)),
          pl.BlockSpec((D, D), lambda i: (0, 0)),
      ],
      out_specs=pl.BlockSpec((_BR, D), lambda i: (i, 0)),
      out_shape=jax.ShapeDtypeStruct((N, D), jnp.float32),
  )(g, p, b, w)


def _tc_final(g, p, b, w, bo):
  return pl.pallas_call(
      _final_body,
      grid=(_GRID,),
      in_specs=[
          pl.BlockSpec((_BR, D), lambda i: (i, 0)),
          pl.BlockSpec((2, _BR, D), lambda i: (0, i, 0)),
          pl.BlockSpec((1, D), lambda i: (0, 0)),
          pl.BlockSpec((D, D), lambda i: (0, 0)),
          pl.BlockSpec((1, D), lambda i: (0, 0)),
      ],
      out_specs=pl.BlockSpec((_BR, D), lambda i: (i, 0)),
      out_shape=jax.ShapeDtypeStruct((N, D), jnp.float32),
  )(g, p, b, w, bo)


@jax.jit
def kernel(x, edge_index, W1, b1, W2, b2, Wo, bo):
  row = edge_index[0]
  col = edge_index[1]
  rowp = jnp.pad(row, (0, _EPAD - E))
  colp = jnp.pad(col, (0, _EPAD - E), constant_values=N)
  row_parts = []
  col_parts = []
  off = 0
  for w in range(NW):
    sz = _SIZES[w]
    tail = CHM * CHUNK - sz
    row_parts.append(jnp.pad(lax.slice(rowp, (off,), (off + sz,)), (0, tail)))
    col_parts.append(jnp.pad(lax.slice(colp, (off,), (off + sz,)), (0, tail),
                             constant_values=N))
    off += sz
  row3 = jnp.stack(row_parts).reshape(NW, CHM, CHUNK)
  col3 = jnp.stack(col_parts).reshape(NW, CHM, CHUNK)

  b1r = b1.reshape(1, D)
  b2r = b2.reshape(1, D)
  bor = bo.reshape(1, D)

  g1 = _tc_matmul(x, W1)
  p1 = _sc_neighbor_sum(g1, row3, col3)
  g2 = _tc_fuse_matmul(g1, p1, b1r, W2)
  p2 = _sc_neighbor_sum(g2, row3, col3)
  return _tc_final(g2, p2, b2r, Wo, bor)


# speedup vs baseline: 1.0505x; 1.0171x over previous
import functools

import jax
import jax.numpy as jnp
from jax import lax
from jax.experimental import pallas as pl
from jax.experimental.pallas import tpu as pltpu
from jax.experimental.pallas import tpu_sc as plsc

N = 10000
E = 320000
D = 128

NC = 2
NS = 16
NW = NC * NS
CHUNK = 128
CF = 120
CS = 38
FAST_CID = 0
CHM = CF
EP = NW * CHM * CHUNK
N_ACC = 10240
ZROWS = N_ACC // NS
ZITER = ZROWS // CHUNK
_SIZES = [CHUNK * (CF if (w % NC) == FAST_CID else CS) for w in range(NW)]
_EPAD = sum(_SIZES)


def _sc_neighbor_sum(g, row3, col3):
  mesh = plsc.VectorSubcoreMesh(core_axis_name="c", subcore_axis_name="s")

  @functools.partial(
      pl.kernel,
      out_type=jax.ShapeDtypeStruct((NC, N_ACC, D), jnp.float32),
      mesh=mesh,
      scratch_types=[
          pltpu.VMEM((CHM, CHUNK), jnp.int32),
          pltpu.VMEM((CHUNK,), jnp.int32),
          pltpu.VMEM((CHUNK,), jnp.int32),
          pltpu.VMEM((CHUNK, D), jnp.float32),
          pltpu.VMEM((CHUNK, D), jnp.float32),
          pltpu.VMEM_SHARED((N_ACC, D), jnp.float32),
          pltpu.SemaphoreType.DMA,
          pltpu.SemaphoreType.DMA,
          pltpu.SemaphoreType.DMA,
          pltpu.SemaphoreType.DMA,
      ],
  )
  def k(g_hbm, row_hbm, col_hbm, out_hbm, rowv, cb, cb2, buf, buf2, acc,
        sem, sem2, csem, csem2):
    cid = lax.axis_index("c")
    sid = lax.axis_index("s")
    wid = sid * NC + cid
    nch = jnp.where(cid == FAST_CID, CF, CS)

    pltpu.async_copy(row_hbm.at[wid], rowv, csem)

    @pl.loop(0, CHUNK)
    def _(r):
      for c in range(0, D, 16):
        buf[r, pl.ds(c, 16)] = jnp.zeros((16,), jnp.float32)

    @pl.loop(0, ZITER)
    def _(z):
      pltpu.sync_copy(buf, acc.at[pl.ds(sid * ZROWS + z * CHUNK, CHUNK)])

    pltpu.make_async_copy(row_hbm.at[wid], rowv, csem).wait()
    plsc.subcore_barrier()

    @pl.when(nch > 0)
    def _():
      pltpu.async_copy(g_hbm.at[rowv.at[0]], buf, sem)
      pltpu.async_copy(col_hbm.at[wid, 0], cb, csem)

    @pl.loop(0, nch, step=2)
    def _(j):
      @pl.when(j + 1 < nch)
      def _():
        pltpu.async_copy(g_hbm.at[rowv.at[j + 1]], buf2, sem2)
        pltpu.async_copy(col_hbm.at[wid, j + 1], cb2, csem2)

      pltpu.make_async_copy(g_hbm.at[rowv.at[j]], buf, sem).wait()
      pltpu.make_async_copy(col_hbm.at[wid, j], cb, csem).wait()
      pltpu.sync_copy(buf, acc.at[cb], add=True)

      @pl.when(j + 2 < nch)
      def _():
        pltpu.async_copy(g_hbm.at[rowv.at[j + 2]], buf, sem)
        pltpu.async_copy(col_hbm.at[wid, j + 2], cb, csem)

      @pl.when(j + 1 < nch)
      def _():
        pltpu.make_async_copy(g_hbm.at[rowv.at[j + 1]], buf2, sem2).wait()
        pltpu.make_async_copy(col_hbm.at[wid, j + 1], cb2, csem2).wait()
        pltpu.sync_copy(buf2, acc.at[cb2], add=True)

    plsc.subcore_barrier()

    @pl.loop(0, ZITER)
    def _(z):
      b = sid * ZROWS + z * CHUNK
      pltpu.sync_copy(acc.at[pl.ds(b, CHUNK)], out_hbm.at[cid, pl.ds(b, CHUNK)])

  return k(g, row3, col3)


_BR = 2000
_GRID = N // _BR


def _mm_body(x_ref, w_ref, o_ref):
  o_ref[...] = jnp.dot(x_ref[...], w_ref[...],
                       preferred_element_type=jnp.float32)


def _fuse_body(g_ref, p_ref, b_ref, w_ref, o_ref):
  h = g_ref[...] + p_ref[0] + p_ref[1] + b_ref[...]
  h = jnp.maximum(h, 0.0)
  o_ref[...] = jnp.dot(h, w_ref[...], preferred_element_type=jnp.float32)


def _final_body(g_ref, p_ref, b_ref, w_ref, bo_ref, o_ref):
  h = g_ref[...] + p_ref[0] + p_ref[1] + b_ref[...]
  h = jnp.maximum(h, 0.0)
  t = jnp.dot(h, w_ref[...], preferred_element_type=jnp.float32) + bo_ref[...]
  m = jnp.max(t, axis=1, keepdims=True)
  e = t - m
  o_ref[...] = e - jnp.log(jnp.sum(jnp.exp(e), axis=1, keepdims=True))


def _tc_matmul(x, w):
  return pl.pallas_call(
      _mm_body,
      grid=(_GRID,),
      in_specs=[
          pl.BlockSpec((_BR, D), lambda i: (i, 0)),
          pl.BlockSpec((D, D), lambda i: (0, 0)),
      ],
      out_specs=pl.BlockSpec((_BR, D), lambda i: (i, 0)),
      out_shape=jax.ShapeDtypeStruct((N, D), jnp.float32),
  )(x, w)


def _tc_fuse_matmul(g, p, b, w):
  return pl.pallas_call(
      _fuse_body,
      grid=(_GRID,),
      in_specs=[
          pl.BlockSpec((_BR, D), lambda i: (i, 0)),
          pl.BlockSpec((2, _BR, D), lambda i: (0, i, 0)),
          pl.BlockSpec((1, D), lambda i: (0, 0)),
          pl.BlockSpec((D, D), lambda i: (0, 0)),
      ],
      out_specs=pl.BlockSpec((_BR, D), lambda i: (i, 0)),
      out_shape=jax.ShapeDtypeStruct((N, D), jnp.float32),
  )(g, p, b, w)


def _tc_final(g, p, b, w, bo):
  return pl.pallas_call(
      _final_body,
      grid=(_GRID,),
      in_specs=[
          pl.BlockSpec((_BR, D), lambda i: (i, 0)),
          pl.BlockSpec((2, _BR, D), lambda i: (0, i, 0)),
          pl.BlockSpec((1, D), lambda i: (0, 0)),
          pl.BlockSpec((D, D), lambda i: (0, 0)),
          pl.BlockSpec((1, D), lambda i: (0, 0)),
      ],
      out_specs=pl.BlockSpec((_BR, D), lambda i: (i, 0)),
      out_shape=jax.ShapeDtypeStruct((N, D), jnp.float32),
  )(g, p, b, w, bo)


@jax.jit
def kernel(x, edge_index, W1, b1, W2, b2, Wo, bo):
  row = edge_index[0]
  col = edge_index[1]
  rowp = jnp.pad(row, (0, _EPAD - E))
  colp = jnp.pad(col, (0, _EPAD - E), constant_values=N)
  row_parts = []
  col_parts = []
  off = 0
  for w in range(NW):
    sz = _SIZES[w]
    tail = CHM * CHUNK - sz
    row_parts.append(jnp.pad(lax.slice(rowp, (off,), (off + sz,)), (0, tail)))
    col_parts.append(jnp.pad(lax.slice(colp, (off,), (off + sz,)), (0, tail),
                             constant_values=N))
    off += sz
  row3 = jnp.stack(row_parts).reshape(NW, CHM, CHUNK)
  col3 = jnp.stack(col_parts).reshape(NW, CHM, CHUNK)

  b1r = b1.reshape(1, D)
  b2r = b2.reshape(1, D)
  bor = bo.reshape(1, D)

  g1 = _tc_matmul(x, W1)
  p1 = _sc_neighbor_sum(g1, row3, col3)
  g2 = _tc_fuse_matmul(g1, p1, b1r, W2)
  p2 = _sc_neighbor_sum(g2, row3, col3)
  return _tc_final(g2, p2, b2r, Wo, bor)
